# baseline (device time: 2471987 ns/iter reference)
import jax
import jax.numpy as jnp
from jax import lax
from jax.experimental import pallas as pl
from jax.experimental.pallas import tpu as pltpu

T = 2048
D = 4096
V_HALF = 8192


def _exchange_halves(logits):

    def body(lg_ref, out_ref, local_sem, send_sem, recv_sem):
        my_x = lax.axis_index("x")
        my_y = lax.axis_index("y")
        partner = (my_x, 1 - my_y)

        barrier = pltpu.get_barrier_semaphore()
        pl.semaphore_signal(
            barrier, inc=1, device_id=partner,
            device_id_type=pl.DeviceIdType.MESH,
        )
        pl.semaphore_wait(barrier, 1)

        local = pltpu.make_async_copy(lg_ref, out_ref.at[my_y], local_sem)
        local.start()
        rdma = pltpu.make_async_remote_copy(
            src_ref=lg_ref,
            dst_ref=out_ref.at[my_y],
            send_sem=send_sem,
            recv_sem=recv_sem,
            device_id=partner,
            device_id_type=pl.DeviceIdType.MESH,
        )
        rdma.start()
        local.wait()
        rdma.wait()

    return pl.pallas_call(
        body,
        out_shape=jax.ShapeDtypeStruct((2, T, V_HALF), logits.dtype),
        in_specs=[pl.BlockSpec(memory_space=pl.ANY)],
        out_specs=pl.BlockSpec(memory_space=pl.ANY),
        scratch_shapes=[
            pltpu.SemaphoreType.DMA,
            pltpu.SemaphoreType.DMA,
            pltpu.SemaphoreType.DMA,
        ],
        compiler_params=pltpu.CompilerParams(collective_id=0),
    )(logits)


def kernel(x, W):
    logits = x @ W
    halves = _exchange_halves(logits)
    full = jnp.concatenate([halves[0], halves[1]], axis=-1)
    m = full.max(axis=-1, keepdims=True)
    e = jnp.exp(full - m)
    return (e / e.sum(axis=-1, keepdims=True)).astype(jnp.float32)


# device time: 1795435 ns/iter; 1.3768x vs baseline; 1.3768x over previous
import jax
import jax.numpy as jnp
from jax import lax
from jax.experimental import pallas as pl
from jax.experimental.pallas import tpu as pltpu

T = 2048
D = 4096
V_HALF = 8192


def _exchange(mine):

    def body(src_ref, out_ref, send_sem, recv_sem):
        my_x = lax.axis_index("x")
        my_y = lax.axis_index("y")
        partner = (my_x, 1 - my_y)

        barrier = pltpu.get_barrier_semaphore()
        pl.semaphore_signal(
            barrier, inc=1, device_id=partner,
            device_id_type=pl.DeviceIdType.MESH,
        )
        pl.semaphore_wait(barrier, 1)

        rdma = pltpu.make_async_remote_copy(
            src_ref=src_ref, dst_ref=out_ref,
            send_sem=send_sem, recv_sem=recv_sem,
            device_id=partner, device_id_type=pl.DeviceIdType.MESH,
        )
        rdma.start()
        rdma.wait()

    return pl.pallas_call(
        body,
        out_shape=jax.ShapeDtypeStruct(mine.shape, mine.dtype),
        in_specs=[pl.BlockSpec(memory_space=pl.ANY)],
        out_specs=pl.BlockSpec(memory_space=pl.ANY),
        scratch_shapes=[pltpu.SemaphoreType.DMA, pltpu.SemaphoreType.DMA],
        compiler_params=pltpu.CompilerParams(collective_id=0),
    )(mine)


def kernel(x, W):
    my_y = lax.axis_index("y")
    mine = x @ W
    theirs = _exchange(mine)

    m = jnp.maximum(
        mine.max(axis=-1, keepdims=True), theirs.max(axis=-1, keepdims=True)
    )
    e_mine = jnp.exp(mine - m)
    e_theirs = jnp.exp(theirs - m)
    s = e_mine.sum(axis=-1, keepdims=True) + e_theirs.sum(axis=-1, keepdims=True)
    p_mine = e_mine / s
    p_theirs = e_theirs / s

    out = jnp.empty((T, 2 * V_HALF), jnp.float32)
    out = lax.dynamic_update_slice(out, p_mine, (0, my_y * V_HALF))
    out = lax.dynamic_update_slice(out, p_theirs, (0, (1 - my_y) * V_HALF))
    return out


# device time: 1254639 ns/iter; 1.9703x vs baseline; 1.4310x over previous
import jax
import jax.numpy as jnp
from jax import lax
from jax.experimental import pallas as pl
from jax.experimental.pallas import tpu as pltpu

T = 2048
D = 4096
V_HALF = 8192


def _exchange(mine):

    def body(src_ref, out_ref, send_sem, recv_sem):
        my_x = lax.axis_index("x")
        my_y = lax.axis_index("y")
        partner = (my_x, 1 - my_y)

        barrier = pltpu.get_barrier_semaphore()
        pl.semaphore_signal(
            barrier, inc=1, device_id=partner,
            device_id_type=pl.DeviceIdType.MESH,
        )
        pl.semaphore_wait(barrier, 1)

        rdma = pltpu.make_async_remote_copy(
            src_ref=src_ref, dst_ref=out_ref,
            send_sem=send_sem, recv_sem=recv_sem,
            device_id=partner, device_id_type=pl.DeviceIdType.MESH,
        )
        rdma.start()
        rdma.wait()

    return pl.pallas_call(
        body,
        out_shape=jax.ShapeDtypeStruct(mine.shape, mine.dtype),
        in_specs=[pl.BlockSpec(memory_space=pl.ANY)],
        out_specs=pl.BlockSpec(memory_space=pl.ANY),
        scratch_shapes=[pltpu.SemaphoreType.DMA, pltpu.SemaphoreType.DMA],
        compiler_params=pltpu.CompilerParams(collective_id=0),
    )(mine)


def kernel(x, W):
    my_y = lax.axis_index("y")
    mine = x @ W
    theirs = _exchange(mine)

    full = lax.cond(
        my_y == 0,
        lambda a, b: jnp.concatenate([a, b], axis=-1),
        lambda a, b: jnp.concatenate([b, a], axis=-1),
        mine, theirs,
    )
    m = full.max(axis=-1, keepdims=True)
    e = jnp.exp(full - m)
    return e / e.sum(axis=-1, keepdims=True)


# device time: 681382 ns/iter; 3.6279x vs baseline; 1.8413x over previous
import jax
import jax.numpy as jnp
from jax import lax
from jax.experimental import pallas as pl
from jax.experimental.pallas import tpu as pltpu

T = 2048
D = 4096
V_HALF = 8192
BLK = 64


def _exchange(mine):

    def body(src_ref, out_ref, send_sem, recv_sem):
        my_x = lax.axis_index("x")
        my_y = lax.axis_index("y")
        partner = (my_x, 1 - my_y)

        barrier = pltpu.get_barrier_semaphore()
        pl.semaphore_signal(
            barrier, inc=1, device_id=partner,
            device_id_type=pl.DeviceIdType.MESH,
        )
        pl.semaphore_wait(barrier, 1)

        rdma = pltpu.make_async_remote_copy(
            src_ref=src_ref, dst_ref=out_ref,
            send_sem=send_sem, recv_sem=recv_sem,
            device_id=partner, device_id_type=pl.DeviceIdType.MESH,
        )
        rdma.start()
        rdma.wait()

    return pl.pallas_call(
        body,
        out_shape=jax.ShapeDtypeStruct(mine.shape, mine.dtype),
        in_specs=[pl.BlockSpec(memory_space=pl.ANY)],
        out_specs=pl.BlockSpec(memory_space=pl.ANY),
        scratch_shapes=[pltpu.SemaphoreType.DMA, pltpu.SemaphoreType.DMA],
        compiler_params=pltpu.CompilerParams(collective_id=0),
    )(mine)


def _softmax_assemble(mine, theirs):

    def body(mine_ref, theirs_ref, out_ref):
        my_y = lax.axis_index("y")
        mn = mine_ref[...]
        th = theirs_ref[...].astype(jnp.float32)
        m = jnp.maximum(
            mn.max(axis=-1, keepdims=True), th.max(axis=-1, keepdims=True)
        )
        em = jnp.exp(mn - m)
        et = jnp.exp(th - m)
        s = em.sum(axis=-1, keepdims=True) + et.sum(axis=-1, keepdims=True)
        pm = em / s
        pt = et / s

        @pl.when(my_y == 0)
        def _():
            out_ref[:, :V_HALF] = pm
            out_ref[:, V_HALF:] = pt

        @pl.when(my_y == 1)
        def _():
            out_ref[:, :V_HALF] = pt
            out_ref[:, V_HALF:] = pm

    return pl.pallas_call(
        body,
        out_shape=jax.ShapeDtypeStruct((T, 2 * V_HALF), jnp.float32),
        grid=(T // BLK,),
        in_specs=[
            pl.BlockSpec((BLK, V_HALF), lambda i: (i, 0)),
            pl.BlockSpec((BLK, V_HALF), lambda i: (i, 0)),
        ],
        out_specs=pl.BlockSpec((BLK, 2 * V_HALF), lambda i: (i, 0)),
    )(mine, theirs)


def kernel(x, W):
    mine = x @ W
    theirs = _exchange(mine.astype(jnp.bfloat16))
    return _softmax_assemble(mine, theirs)


# device time: 528528 ns/iter; 4.6771x vs baseline; 1.2892x over previous
import jax
import jax.numpy as jnp
from jax import lax
from jax.experimental import pallas as pl
from jax.experimental.pallas import tpu as pltpu

T = 2048
D = 4096
V_HALF = 8192
BLK = 64


NC = 8
CROWS = T // 2 // NC


def _exchange(mine):

    def body(src_ref, out_ref, send_y, recv_y, send_x, recv_x):
        my_x = lax.axis_index("x")
        my_y = lax.axis_index("y")
        ynbr = (my_x, 1 - my_y)
        xnbr = (1 - my_x, my_y)

        barrier = pltpu.get_barrier_semaphore()
        for nbr in (ynbr, xnbr):
            pl.semaphore_signal(
                barrier, inc=1, device_id=nbr,
                device_id_type=pl.DeviceIdType.MESH,
            )
        pl.semaphore_wait(barrier, 2)

        part0 = my_x * (T // 2)

        directs = []
        for c in range(NC):
            rows = pl.ds(part0 + c * CROWS, CROWS)
            rdma = pltpu.make_async_remote_copy(
                src_ref=src_ref.at[rows], dst_ref=out_ref.at[rows],
                send_sem=send_y.at[c], recv_sem=recv_y.at[c],
                device_id=ynbr, device_id_type=pl.DeviceIdType.MESH,
            )
            rdma.start()
            directs.append(rdma)

        fwds = []
        for c in range(NC):
            directs[c].wait_recv()
            rows = pl.ds(part0 + c * CROWS, CROWS)
            fwd = pltpu.make_async_remote_copy(
                src_ref=out_ref.at[rows], dst_ref=out_ref.at[rows],
                send_sem=send_x.at[c], recv_sem=recv_x.at[c],
                device_id=xnbr, device_id_type=pl.DeviceIdType.MESH,
            )
            fwd.start()
            fwds.append(fwd)

        for c in range(NC):
            fwds[c].wait_recv()
        for c in range(NC):
            directs[c].wait_send()
            fwds[c].wait_send()

    return pl.pallas_call(
        body,
        out_shape=jax.ShapeDtypeStruct(mine.shape, mine.dtype),
        in_specs=[pl.BlockSpec(memory_space=pl.ANY)],
        out_specs=pl.BlockSpec(memory_space=pl.ANY),
        scratch_shapes=[pltpu.SemaphoreType.DMA((NC,))] * 4,
        compiler_params=pltpu.CompilerParams(collective_id=0),
    )(mine)


def _softmax_assemble(mine, theirs):

    def body(mine_ref, theirs_ref, out_ref):
        my_y = lax.axis_index("y")
        mn = mine_ref[...]
        th = theirs_ref[...].astype(jnp.float32)
        m = jnp.maximum(
            mn.max(axis=-1, keepdims=True), th.max(axis=-1, keepdims=True)
        )
        em = jnp.exp(mn - m)
        et = jnp.exp(th - m)
        s = em.sum(axis=-1, keepdims=True) + et.sum(axis=-1, keepdims=True)
        pm = em / s
        pt = et / s

        @pl.when(my_y == 0)
        def _():
            out_ref[:, :V_HALF] = pm
            out_ref[:, V_HALF:] = pt

        @pl.when(my_y == 1)
        def _():
            out_ref[:, :V_HALF] = pt
            out_ref[:, V_HALF:] = pm

    return pl.pallas_call(
        body,
        out_shape=jax.ShapeDtypeStruct((T, 2 * V_HALF), jnp.float32),
        grid=(T // BLK,),
        in_specs=[
            pl.BlockSpec((BLK, V_HALF), lambda i: (i, 0)),
            pl.BlockSpec((BLK, V_HALF), lambda i: (i, 0)),
        ],
        out_specs=pl.BlockSpec((BLK, 2 * V_HALF), lambda i: (i, 0)),
    )(mine, theirs)


def kernel(x, W):
    mine = x @ W
    theirs = _exchange(mine.astype(jnp.bfloat16))
    return _softmax_assemble(mine, theirs)


# device time: 477855 ns/iter; 5.1731x vs baseline; 1.1060x over previous
import jax
import jax.numpy as jnp
from jax import lax
from jax.experimental import pallas as pl
from jax.experimental.pallas import tpu as pltpu

T = 2048
D = 4096
V_HALF = 8192
BLK = 64


NC = 8
CROWS = T // 2 // NC


def _exchange(mine):

    def body(src_ref, out_ref, send_y, recv_y, send_x, recv_x):
        my_x = lax.axis_index("x")
        my_y = lax.axis_index("y")
        ynbr = (my_x, 1 - my_y)
        xnbr = (1 - my_x, my_y)

        barrier = pltpu.get_barrier_semaphore()
        for nbr in (ynbr, xnbr):
            pl.semaphore_signal(
                barrier, inc=1, device_id=nbr,
                device_id_type=pl.DeviceIdType.MESH,
            )
        pl.semaphore_wait(barrier, 2)

        part0 = my_x * (T // 2)

        directs = []
        for c in range(NC):
            rows = pl.ds(part0 + c * CROWS, CROWS)
            rdma = pltpu.make_async_remote_copy(
                src_ref=src_ref.at[rows], dst_ref=out_ref.at[rows],
                send_sem=send_y.at[c], recv_sem=recv_y.at[c],
                device_id=ynbr, device_id_type=pl.DeviceIdType.MESH,
            )
            rdma.start()
            directs.append(rdma)

        fwds = []
        for c in range(NC):
            directs[c].wait_recv()
            rows = pl.ds(part0 + c * CROWS, CROWS)
            fwd = pltpu.make_async_remote_copy(
                src_ref=out_ref.at[rows], dst_ref=out_ref.at[rows],
                send_sem=send_x.at[c], recv_sem=recv_x.at[c],
                device_id=xnbr, device_id_type=pl.DeviceIdType.MESH,
            )
            fwd.start()
            fwds.append(fwd)

        for c in range(NC):
            fwds[c].wait_recv()
        for c in range(NC):
            directs[c].wait_send()
            fwds[c].wait_send()

    return pl.pallas_call(
        body,
        out_shape=jax.ShapeDtypeStruct(mine.shape, mine.dtype),
        in_specs=[pl.BlockSpec(memory_space=pl.ANY)],
        out_specs=pl.BlockSpec(memory_space=pl.ANY),
        scratch_shapes=[pltpu.SemaphoreType.DMA((NC,))] * 4,
        compiler_params=pltpu.CompilerParams(collective_id=0),
    )(mine)


def _softmax_assemble(mine, theirs):

    def body(mine_ref, theirs_ref, out_ref):
        my_y = lax.axis_index("y")
        mn = mine_ref[...].astype(jnp.float32)
        th = theirs_ref[...].astype(jnp.float32)
        m = jnp.maximum(
            mn.max(axis=-1, keepdims=True), th.max(axis=-1, keepdims=True)
        )
        em = jnp.exp(mn - m)
        et = jnp.exp(th - m)
        s = em.sum(axis=-1, keepdims=True) + et.sum(axis=-1, keepdims=True)
        pm = em / s
        pt = et / s

        @pl.when(my_y == 0)
        def _():
            out_ref[:, :V_HALF] = pm
            out_ref[:, V_HALF:] = pt

        @pl.when(my_y == 1)
        def _():
            out_ref[:, :V_HALF] = pt
            out_ref[:, V_HALF:] = pm

    return pl.pallas_call(
        body,
        out_shape=jax.ShapeDtypeStruct((T, 2 * V_HALF), jnp.float32),
        grid=(T // BLK,),
        in_specs=[
            pl.BlockSpec((BLK, V_HALF), lambda i: (i, 0)),
            pl.BlockSpec((BLK, V_HALF), lambda i: (i, 0)),
        ],
        out_specs=pl.BlockSpec((BLK, 2 * V_HALF), lambda i: (i, 0)),
    )(mine, theirs)


def kernel(x, W):
    mine = jnp.dot(
        x.astype(jnp.bfloat16), W.astype(jnp.bfloat16),
        preferred_element_type=jnp.bfloat16,
    )
    theirs = _exchange(mine)
    return _softmax_assemble(mine, theirs)


# device time: 467041 ns/iter; 5.2929x vs baseline; 1.0232x over previous
import jax
import jax.numpy as jnp
from jax import lax
from jax.experimental import pallas as pl
from jax.experimental.pallas import tpu as pltpu

T = 2048
D = 4096
V_HALF = 8192
BLK = 64


NC = 16
CROWS = T // 2 // NC


def _exchange(mine):

    def body(src_ref, out_ref, send_y, recv_y, send_x, recv_x):
        my_x = lax.axis_index("x")
        my_y = lax.axis_index("y")
        ynbr = (my_x, 1 - my_y)
        xnbr = (1 - my_x, my_y)

        barrier = pltpu.get_barrier_semaphore()
        for nbr in (ynbr, xnbr):
            pl.semaphore_signal(
                barrier, inc=1, device_id=nbr,
                device_id_type=pl.DeviceIdType.MESH,
            )
        pl.semaphore_wait(barrier, 2)

        part0 = my_x * (T // 2)

        directs = []
        for c in range(NC):
            rows = pl.ds(part0 + c * CROWS, CROWS)
            rdma = pltpu.make_async_remote_copy(
                src_ref=src_ref.at[rows], dst_ref=out_ref.at[rows],
                send_sem=send_y.at[c], recv_sem=recv_y.at[c],
                device_id=ynbr, device_id_type=pl.DeviceIdType.MESH,
            )
            rdma.start()
            directs.append(rdma)

        fwds = []
        for c in range(NC):
            directs[c].wait_recv()
            rows = pl.ds(part0 + c * CROWS, CROWS)
            fwd = pltpu.make_async_remote_copy(
                src_ref=out_ref.at[rows], dst_ref=out_ref.at[rows],
                send_sem=send_x.at[c], recv_sem=recv_x.at[c],
                device_id=xnbr, device_id_type=pl.DeviceIdType.MESH,
            )
            fwd.start()
            fwds.append(fwd)

        for c in range(NC):
            fwds[c].wait_recv()
        for c in range(NC):
            directs[c].wait_send()
            fwds[c].wait_send()

    return pl.pallas_call(
        body,
        out_shape=jax.ShapeDtypeStruct(mine.shape, mine.dtype),
        in_specs=[pl.BlockSpec(memory_space=pl.ANY)],
        out_specs=pl.BlockSpec(memory_space=pl.ANY),
        scratch_shapes=[pltpu.SemaphoreType.DMA((NC,))] * 4,
        compiler_params=pltpu.CompilerParams(collective_id=0),
    )(mine)


def _softmax_assemble(mine, theirs):

    def body(mine_ref, theirs_ref, out_ref):
        my_y = lax.axis_index("y")
        mn = mine_ref[...].astype(jnp.float32)
        th = theirs_ref[...].astype(jnp.float32)
        m = jnp.maximum(
            mn.max(axis=-1, keepdims=True), th.max(axis=-1, keepdims=True)
        )
        em = jnp.exp(mn - m)
        et = jnp.exp(th - m)
        s = em.sum(axis=-1, keepdims=True) + et.sum(axis=-1, keepdims=True)
        pm = em / s
        pt = et / s

        @pl.when(my_y == 0)
        def _():
            out_ref[:, :V_HALF] = pm
            out_ref[:, V_HALF:] = pt

        @pl.when(my_y == 1)
        def _():
            out_ref[:, :V_HALF] = pt
            out_ref[:, V_HALF:] = pm

    return pl.pallas_call(
        body,
        out_shape=jax.ShapeDtypeStruct((T, 2 * V_HALF), jnp.float32),
        grid=(T // BLK,),
        in_specs=[
            pl.BlockSpec((BLK, V_HALF), lambda i: (i, 0)),
            pl.BlockSpec((BLK, V_HALF), lambda i: (i, 0)),
        ],
        out_specs=pl.BlockSpec((BLK, 2 * V_HALF), lambda i: (i, 0)),
    )(mine, theirs)


def kernel(x, W):
    mine = jnp.dot(
        x.astype(jnp.bfloat16), W.astype(jnp.bfloat16),
        preferred_element_type=jnp.bfloat16,
    )
    theirs = _exchange(mine)
    return _softmax_assemble(mine, theirs)
